# Initial kernel scaffold; baseline (speedup 1.0000x reference)
import jax
import jax.numpy as jnp
from jax.experimental import pallas as pl


def kernel(inputs, edge_index, W1_self, W1_neigh, b1, W2_self, W2_neigh, b2):
    raise NotImplementedError("write your pallas kernel here")


# speedup vs baseline: 2.7478x; 2.7478x over previous
import functools

import jax
import jax.numpy as jnp
from jax import lax
from jax.experimental import pallas as pl
from jax.experimental.pallas import tpu as pltpu
from jax.experimental.pallas import tpu_sc as plsc

N_NODES = 10000
D = 128

NC = 2
NS = 16
NW = NC * NS

BLK = 128
BLOCKS_PER_WORKER = 80
EDGES_PER_WORKER = BLK * BLOCKS_PER_WORKER
E_PAD = EDGES_PER_WORKER * NW
N_PAD = 10240
ROWS_PER_SUBCORE = N_PAD // NS

_MESH = plsc.VectorSubcoreMesh(core_axis_name="c", subcore_axis_name="s",
                               num_cores=NC, num_subcores=NS)


def _sc_aggregate(h, src_p, dst_p, zeros_rows):
    @pl.kernel(out_type=jax.ShapeDtypeStruct((NC, N_PAD, D), jnp.float32),
               mesh=_MESH,
               scratch_types=(pltpu.VMEM((BLK,), jnp.int32),
                              pltpu.VMEM((BLK,), jnp.int32),
                              pltpu.VMEM((BLK, D), jnp.float32),
                              pltpu.VMEM_SHARED((N_PAD, D), jnp.float32)))
    def k(h_hbm, src_hbm, dst_hbm, z_hbm, out_hbm, src_v, dst_v, rows_v, agg_sh):
        cid = lax.axis_index("c")
        sid = lax.axis_index("s")
        wid = sid * NC + cid
        row0 = sid * ROWS_PER_SUBCORE
        pltpu.sync_copy(z_hbm, agg_sh.at[pl.ds(row0, ROWS_PER_SUBCORE)])
        plsc.subcore_barrier()
        base = wid * EDGES_PER_WORKER

        @pl.loop(0, BLOCKS_PER_WORKER)
        def _(b):
            off = base + b * BLK
            pltpu.sync_copy(src_hbm.at[pl.ds(off, BLK)], src_v)
            pltpu.sync_copy(dst_hbm.at[pl.ds(off, BLK)], dst_v)
            pltpu.sync_copy(h_hbm.at[src_v], rows_v)
            pltpu.sync_copy(rows_v, agg_sh.at[dst_v], add=True)

        plsc.subcore_barrier()
        pltpu.sync_copy(agg_sh.at[pl.ds(row0, ROWS_PER_SUBCORE)],
                        out_hbm.at[cid, pl.ds(row0, ROWS_PER_SUBCORE)])

    return k(h, src_p, dst_p, zeros_rows)


def _sc_degree(dst_p, ones_rows, zeros_rows):
    @pl.kernel(out_type=jax.ShapeDtypeStruct((NC, N_PAD, D), jnp.float32),
               mesh=_MESH,
               scratch_types=(pltpu.VMEM((BLK,), jnp.int32),
                              pltpu.VMEM((BLK, D), jnp.float32),
                              pltpu.VMEM_SHARED((N_PAD, D), jnp.float32)))
    def k(dst_hbm, ones_hbm, z_hbm, out_hbm, dst_v, ones_v, deg_sh):
        cid = lax.axis_index("c")
        sid = lax.axis_index("s")
        wid = sid * NC + cid
        row0 = sid * ROWS_PER_SUBCORE
        pltpu.sync_copy(z_hbm, deg_sh.at[pl.ds(row0, ROWS_PER_SUBCORE)])
        pltpu.sync_copy(ones_hbm, ones_v)
        plsc.subcore_barrier()
        base = wid * EDGES_PER_WORKER

        @pl.loop(0, BLOCKS_PER_WORKER)
        def _(b):
            off = base + b * BLK
            pltpu.sync_copy(dst_hbm.at[pl.ds(off, BLK)], dst_v)
            pltpu.sync_copy(ones_v, deg_sh.at[dst_v], add=True)

        plsc.subcore_barrier()
        pltpu.sync_copy(deg_sh.at[pl.ds(row0, ROWS_PER_SUBCORE)],
                        out_hbm.at[cid, pl.ds(row0, ROWS_PER_SUBCORE)])

    return k(dst_p, ones_rows, zeros_rows)


ROW_BLK = 1000


def _tc_layer_body(apply_relu, x_ref, a_ref, d_ref, ws_ref, wn_ref, b_ref, o_ref):
    inv = 1.0 / jnp.maximum(d_ref[:, 0] + d_ref[:, 1], 1.0)
    h_neigh = (a_ref[0] + a_ref[1]) * inv[:, None]
    dn = (((1,), (1,)), ((), ()))
    out = (lax.dot_general(x_ref[...], ws_ref[...], dn,
                           preferred_element_type=jnp.float32)
           + lax.dot_general(h_neigh, wn_ref[...], dn,
                             preferred_element_type=jnp.float32)
           + b_ref[...])
    if apply_relu:
        out = jnp.maximum(out, 0.0)
    o_ref[...] = out


def _tc_layer(x, agg_parts, deg, w_self, w_neigh, b, apply_relu):
    n = x.shape[0]
    grid = (n // ROW_BLK,)
    return pl.pallas_call(
        functools.partial(_tc_layer_body, apply_relu),
        grid=grid,
        in_specs=[
            pl.BlockSpec((ROW_BLK, D), lambda i: (i, 0)),
            pl.BlockSpec((NC, ROW_BLK, D), lambda i: (0, i, 0)),
            pl.BlockSpec((ROW_BLK, NC), lambda i: (i, 0)),
            pl.BlockSpec((D, D), lambda i: (0, 0)),
            pl.BlockSpec((D, D), lambda i: (0, 0)),
            pl.BlockSpec((1, D), lambda i: (0, 0)),
        ],
        out_specs=pl.BlockSpec((ROW_BLK, D), lambda i: (i, 0)),
        out_shape=jax.ShapeDtypeStruct((n, D), jnp.float32),
    )(x, agg_parts, deg, w_self, w_neigh, b)


def kernel(inputs, edge_index, W1_self, W1_neigh, b1, W2_self, W2_neigh, b2):
    x = inputs.astype(jnp.float32)
    src = edge_index[0].astype(jnp.int32)
    dst = edge_index[1].astype(jnp.int32)

    n_dummy = E_PAD - src.shape[0]
    src_p = jnp.concatenate([src, jnp.zeros((n_dummy,), jnp.int32)])
    dst_p = jnp.concatenate(
        [dst, N_NODES + jnp.arange(n_dummy, dtype=jnp.int32) % (N_PAD - N_NODES)])

    zeros_rows = jnp.zeros((ROWS_PER_SUBCORE, D), jnp.float32)
    ones_rows = jnp.ones((BLK, D), jnp.float32)
    b1r = b1.reshape(1, D)
    b2r = b2.reshape(1, D)

    deg_parts = _sc_degree(dst_p, ones_rows, zeros_rows)
    deg = deg_parts[:, :, 0].T

    agg1 = _sc_aggregate(x, src_p, dst_p, zeros_rows)
    h = _tc_layer(x, agg1, deg, W1_self, W1_neigh, b1r, apply_relu=True)
    agg2 = _sc_aggregate(h, src_p, dst_p, zeros_rows)
    out = _tc_layer(h, agg2, deg, W2_self, W2_neigh, b2r, apply_relu=False)
    return out


# speedup vs baseline: 3.4105x; 1.2412x over previous
import functools

import jax
import jax.numpy as jnp
from jax import lax
from jax.experimental import pallas as pl
from jax.experimental.pallas import tpu as pltpu
from jax.experimental.pallas import tpu_sc as plsc

N_NODES = 10000
D = 128

NC = 2
NS = 16
NW = NC * NS

BLK = 128
BLOCKS_PER_WORKER = 80
EDGES_PER_WORKER = BLK * BLOCKS_PER_WORKER
E_PAD = EDGES_PER_WORKER * NW
N_PAD = 10240
ROWS_PER_SUBCORE = N_PAD // NS

_MESH = plsc.VectorSubcoreMesh(core_axis_name="c", subcore_axis_name="s",
                               num_cores=NC, num_subcores=NS)


def _sc_aggregate(h, src_p, dst_p, zeros_rows):
    @pl.kernel(out_type=jax.ShapeDtypeStruct((NC, N_PAD, D), jnp.float32),
               mesh=_MESH,
               scratch_types=(pltpu.VMEM((BLOCKS_PER_WORKER, BLK), jnp.int32),
                              pltpu.VMEM((BLOCKS_PER_WORKER, BLK), jnp.int32),
                              pltpu.VMEM((BLK, D), jnp.float32),
                              pltpu.VMEM_SHARED((N_PAD, D), jnp.float32)))
    def k(h_hbm, src_hbm, dst_hbm, z_hbm, out_hbm, src_i, dst_i, rows_v, agg_sh):
        cid = lax.axis_index("c")
        sid = lax.axis_index("s")
        wid = sid * NC + cid
        row0 = sid * ROWS_PER_SUBCORE
        pltpu.sync_copy(z_hbm, agg_sh.at[pl.ds(row0, ROWS_PER_SUBCORE)])
        blk0 = wid * BLOCKS_PER_WORKER
        pltpu.sync_copy(src_hbm.at[pl.ds(blk0, BLOCKS_PER_WORKER)], src_i)
        pltpu.sync_copy(dst_hbm.at[pl.ds(blk0, BLOCKS_PER_WORKER)], dst_i)
        plsc.subcore_barrier()

        @pl.loop(0, BLOCKS_PER_WORKER)
        def _(b):
            pltpu.sync_copy(h_hbm.at[src_i.at[b]], rows_v)
            pltpu.sync_copy(rows_v, agg_sh.at[dst_i.at[b]], add=True)

        plsc.subcore_barrier()
        pltpu.sync_copy(agg_sh.at[pl.ds(row0, ROWS_PER_SUBCORE)],
                        out_hbm.at[cid, pl.ds(row0, ROWS_PER_SUBCORE)])

    return k(h, src_p, dst_p, zeros_rows)


def _sc_degree(dst_p, ones_rows, zeros_rows):
    @pl.kernel(out_type=jax.ShapeDtypeStruct((NC, N_PAD, D), jnp.float32),
               mesh=_MESH,
               scratch_types=(pltpu.VMEM((BLOCKS_PER_WORKER, BLK), jnp.int32),
                              pltpu.VMEM((BLK, D), jnp.float32),
                              pltpu.VMEM_SHARED((N_PAD, D), jnp.float32)))
    def k(dst_hbm, ones_hbm, z_hbm, out_hbm, dst_i, ones_v, deg_sh):
        cid = lax.axis_index("c")
        sid = lax.axis_index("s")
        wid = sid * NC + cid
        row0 = sid * ROWS_PER_SUBCORE
        pltpu.sync_copy(z_hbm, deg_sh.at[pl.ds(row0, ROWS_PER_SUBCORE)])
        pltpu.sync_copy(ones_hbm, ones_v)
        blk0 = wid * BLOCKS_PER_WORKER
        pltpu.sync_copy(dst_hbm.at[pl.ds(blk0, BLOCKS_PER_WORKER)], dst_i)
        plsc.subcore_barrier()

        @pl.loop(0, BLOCKS_PER_WORKER)
        def _(b):
            pltpu.sync_copy(ones_v, deg_sh.at[dst_i.at[b]], add=True)

        plsc.subcore_barrier()
        pltpu.sync_copy(deg_sh.at[pl.ds(row0, ROWS_PER_SUBCORE)],
                        out_hbm.at[cid, pl.ds(row0, ROWS_PER_SUBCORE)])

    return k(dst_p, ones_rows, zeros_rows)


ROW_BLK = 1000


def _tc_layer_body(apply_relu, x_ref, a_ref, d_ref, ws_ref, wn_ref, b_ref, o_ref):
    inv = 1.0 / jnp.maximum(d_ref[:, 0] + d_ref[:, 1], 1.0)
    h_neigh = (a_ref[0] + a_ref[1]) * inv[:, None]
    dn = (((1,), (1,)), ((), ()))
    out = (lax.dot_general(x_ref[...], ws_ref[...], dn,
                           preferred_element_type=jnp.float32)
           + lax.dot_general(h_neigh, wn_ref[...], dn,
                             preferred_element_type=jnp.float32)
           + b_ref[...])
    if apply_relu:
        out = jnp.maximum(out, 0.0)
    o_ref[...] = out


def _tc_layer(x, agg_parts, deg, w_self, w_neigh, b, apply_relu):
    n = x.shape[0]
    grid = (n // ROW_BLK,)
    return pl.pallas_call(
        functools.partial(_tc_layer_body, apply_relu),
        grid=grid,
        in_specs=[
            pl.BlockSpec((ROW_BLK, D), lambda i: (i, 0)),
            pl.BlockSpec((NC, ROW_BLK, D), lambda i: (0, i, 0)),
            pl.BlockSpec((ROW_BLK, NC), lambda i: (i, 0)),
            pl.BlockSpec((D, D), lambda i: (0, 0)),
            pl.BlockSpec((D, D), lambda i: (0, 0)),
            pl.BlockSpec((1, D), lambda i: (0, 0)),
        ],
        out_specs=pl.BlockSpec((ROW_BLK, D), lambda i: (i, 0)),
        out_shape=jax.ShapeDtypeStruct((n, D), jnp.float32),
    )(x, agg_parts, deg, w_self, w_neigh, b)


def kernel(inputs, edge_index, W1_self, W1_neigh, b1, W2_self, W2_neigh, b2):
    x = inputs.astype(jnp.float32)
    src = edge_index[0].astype(jnp.int32)
    dst = edge_index[1].astype(jnp.int32)

    n_dummy = E_PAD - src.shape[0]
    src_p = jnp.concatenate([src, jnp.zeros((n_dummy,), jnp.int32)]).reshape(
        NW * BLOCKS_PER_WORKER, BLK)
    dst_p = jnp.concatenate(
        [dst, N_NODES + jnp.arange(n_dummy, dtype=jnp.int32) % (N_PAD - N_NODES)]
    ).reshape(NW * BLOCKS_PER_WORKER, BLK)

    zeros_rows = jnp.zeros((ROWS_PER_SUBCORE, D), jnp.float32)
    ones_rows = jnp.ones((BLK, D), jnp.float32)
    b1r = b1.reshape(1, D)
    b2r = b2.reshape(1, D)

    deg_parts = _sc_degree(dst_p, ones_rows, zeros_rows)
    deg = deg_parts[:, :, 0].T

    agg1 = _sc_aggregate(x, src_p, dst_p, zeros_rows)
    h = _tc_layer(x, agg1, deg, W1_self, W1_neigh, b1r, apply_relu=True)
    agg2 = _sc_aggregate(h, src_p, dst_p, zeros_rows)
    out = _tc_layer(h, agg2, deg, W2_self, W2_neigh, b2r, apply_relu=False)
    return out


# speedup vs baseline: 3.8031x; 1.1151x over previous
import functools

import jax
import jax.numpy as jnp
from jax import lax
from jax.experimental import pallas as pl
from jax.experimental.pallas import tpu as pltpu
from jax.experimental.pallas import tpu_sc as plsc

N_NODES = 10000
D = 128

NC = 2
NS = 16
NW = NC * NS

BLK = 128
BLOCKS_PER_WORKER = 80
EDGES_PER_WORKER = BLK * BLOCKS_PER_WORKER
E_PAD = EDGES_PER_WORKER * NW
N_PAD = 10240
ROWS_PER_SUBCORE = N_PAD // NS

_MESH = plsc.VectorSubcoreMesh(core_axis_name="c", subcore_axis_name="s",
                               num_cores=NC, num_subcores=NS)


def _sc_aggregate(h, src_p, dst_p, zeros_rows):
    half = BLOCKS_PER_WORKER // 2

    @pl.kernel(out_type=jax.ShapeDtypeStruct((NC, N_PAD, D), jnp.float32),
               mesh=_MESH,
               scratch_types=(pltpu.VMEM((half, BLK), jnp.int32),
                              pltpu.VMEM((half, BLK), jnp.int32),
                              pltpu.VMEM((BLK, D), jnp.float32),
                              pltpu.VMEM((BLK, D), jnp.float32),
                              pltpu.SemaphoreType.DMA,
                              pltpu.SemaphoreType.DMA,
                              pltpu.VMEM_SHARED((N_PAD, D), jnp.float32)))
    def k(h_hbm, src_hbm, dst_hbm, z_hbm, out_hbm, src_i, dst_i,
          rows_a, rows_b, sem_a, sem_b, agg_sh):
        cid = lax.axis_index("c")
        sid = lax.axis_index("s")
        wid = sid * NC + cid
        row0 = sid * ROWS_PER_SUBCORE
        pltpu.sync_copy(z_hbm, agg_sh.at[pl.ds(row0, ROWS_PER_SUBCORE)])
        plsc.subcore_barrier()

        def start(b, rows, sem):
            pltpu.async_copy(h_hbm.at[src_i.at[b]], rows, sem)

        def finish(b, rows, sem):
            pltpu.make_async_copy(h_hbm.at[src_i.at[b]], rows, sem).wait()
            pltpu.sync_copy(rows, agg_sh.at[dst_i.at[b]], add=True)

        for chunk in range(2):
            blk0 = wid * BLOCKS_PER_WORKER + chunk * half
            pltpu.sync_copy(src_hbm.at[pl.ds(blk0, half)], src_i)
            pltpu.sync_copy(dst_hbm.at[pl.ds(blk0, half)], dst_i)
            start(0, rows_a, sem_a)

            @pl.loop(0, half // 2 - 1)
            def _(i):
                b0 = 2 * i
                start(b0 + 1, rows_b, sem_b)
                finish(b0, rows_a, sem_a)
                start(b0 + 2, rows_a, sem_a)
                finish(b0 + 1, rows_b, sem_b)

            start(half - 1, rows_b, sem_b)
            finish(half - 2, rows_a, sem_a)
            finish(half - 1, rows_b, sem_b)

        plsc.subcore_barrier()
        pltpu.sync_copy(agg_sh.at[pl.ds(row0, ROWS_PER_SUBCORE)],
                        out_hbm.at[cid, pl.ds(row0, ROWS_PER_SUBCORE)])

    return k(h, src_p, dst_p, zeros_rows)


def _sc_degree(dst_p, ones_rows, zeros_rows):
    @pl.kernel(out_type=jax.ShapeDtypeStruct((NC, N_PAD, D), jnp.float32),
               mesh=_MESH,
               scratch_types=(pltpu.VMEM((BLOCKS_PER_WORKER, BLK), jnp.int32),
                              pltpu.VMEM((BLK, D), jnp.float32),
                              pltpu.VMEM_SHARED((N_PAD, D), jnp.float32)))
    def k(dst_hbm, ones_hbm, z_hbm, out_hbm, dst_i, ones_v, deg_sh):
        cid = lax.axis_index("c")
        sid = lax.axis_index("s")
        wid = sid * NC + cid
        row0 = sid * ROWS_PER_SUBCORE
        pltpu.sync_copy(z_hbm, deg_sh.at[pl.ds(row0, ROWS_PER_SUBCORE)])
        pltpu.sync_copy(ones_hbm, ones_v)
        blk0 = wid * BLOCKS_PER_WORKER
        pltpu.sync_copy(dst_hbm.at[pl.ds(blk0, BLOCKS_PER_WORKER)], dst_i)
        plsc.subcore_barrier()

        @pl.loop(0, BLOCKS_PER_WORKER)
        def _(b):
            pltpu.sync_copy(ones_v, deg_sh.at[dst_i.at[b]], add=True)

        plsc.subcore_barrier()
        pltpu.sync_copy(deg_sh.at[pl.ds(row0, ROWS_PER_SUBCORE)],
                        out_hbm.at[cid, pl.ds(row0, ROWS_PER_SUBCORE)])

    return k(dst_p, ones_rows, zeros_rows)


ROW_BLK = 1000


def _tc_layer_body(apply_relu, x_ref, a_ref, d_ref, ws_ref, wn_ref, b_ref, o_ref):
    inv = 1.0 / jnp.maximum(d_ref[:, 0] + d_ref[:, 1], 1.0)
    h_neigh = (a_ref[0] + a_ref[1]) * inv[:, None]
    dn = (((1,), (1,)), ((), ()))
    out = (lax.dot_general(x_ref[...], ws_ref[...], dn,
                           preferred_element_type=jnp.float32)
           + lax.dot_general(h_neigh, wn_ref[...], dn,
                             preferred_element_type=jnp.float32)
           + b_ref[...])
    if apply_relu:
        out = jnp.maximum(out, 0.0)
    o_ref[...] = out


def _tc_layer(x, agg_parts, deg, w_self, w_neigh, b, apply_relu):
    n = x.shape[0]
    grid = (n // ROW_BLK,)
    return pl.pallas_call(
        functools.partial(_tc_layer_body, apply_relu),
        grid=grid,
        in_specs=[
            pl.BlockSpec((ROW_BLK, D), lambda i: (i, 0)),
            pl.BlockSpec((NC, ROW_BLK, D), lambda i: (0, i, 0)),
            pl.BlockSpec((ROW_BLK, NC), lambda i: (i, 0)),
            pl.BlockSpec((D, D), lambda i: (0, 0)),
            pl.BlockSpec((D, D), lambda i: (0, 0)),
            pl.BlockSpec((1, D), lambda i: (0, 0)),
        ],
        out_specs=pl.BlockSpec((ROW_BLK, D), lambda i: (i, 0)),
        out_shape=jax.ShapeDtypeStruct((n, D), jnp.float32),
    )(x, agg_parts, deg, w_self, w_neigh, b)


def kernel(inputs, edge_index, W1_self, W1_neigh, b1, W2_self, W2_neigh, b2):
    x = inputs.astype(jnp.float32)
    src = edge_index[0].astype(jnp.int32)
    dst = edge_index[1].astype(jnp.int32)

    n_dummy = E_PAD - src.shape[0]
    src_p = jnp.concatenate([src, jnp.zeros((n_dummy,), jnp.int32)]).reshape(
        NW * BLOCKS_PER_WORKER, BLK)
    dst_p = jnp.concatenate(
        [dst, N_NODES + jnp.arange(n_dummy, dtype=jnp.int32) % (N_PAD - N_NODES)]
    ).reshape(NW * BLOCKS_PER_WORKER, BLK)

    zeros_rows = jnp.zeros((ROWS_PER_SUBCORE, D), jnp.float32)
    ones_rows = jnp.ones((BLK, D), jnp.float32)
    b1r = b1.reshape(1, D)
    b2r = b2.reshape(1, D)

    deg_parts = _sc_degree(dst_p, ones_rows, zeros_rows)
    deg = deg_parts[:, :, 0].T

    agg1 = _sc_aggregate(x, src_p, dst_p, zeros_rows)
    h = _tc_layer(x, agg1, deg, W1_self, W1_neigh, b1r, apply_relu=True)
    agg2 = _sc_aggregate(h, src_p, dst_p, zeros_rows)
    out = _tc_layer(h, agg2, deg, W2_self, W2_neigh, b2r, apply_relu=False)
    return out


# speedup vs baseline: 3.9673x; 1.0432x over previous
import functools

import jax
import jax.numpy as jnp
from jax import lax
from jax.experimental import pallas as pl
from jax.experimental.pallas import tpu as pltpu
from jax.experimental.pallas import tpu_sc as plsc

N_NODES = 10000
D = 128

NC = 2
NS = 16
NW = NC * NS

BLK = 128
BLOCKS_PER_WORKER = 80
EDGES_PER_WORKER = BLK * BLOCKS_PER_WORKER
E_PAD = EDGES_PER_WORKER * NW
N_PAD = 10240
ROWS_PER_SUBCORE = N_PAD // NS

_MESH = plsc.VectorSubcoreMesh(core_axis_name="c", subcore_axis_name="s",
                               num_cores=NC, num_subcores=NS)


K0 = 128
K1 = 32
CHUNK = 32


def _sc_aggregate(h, src_p, dst_p, zeros_rows):
    @pl.kernel(out_type=jax.ShapeDtypeStruct((NC, N_PAD, D), jnp.float32),
               mesh=_MESH,
               scratch_types=(pltpu.VMEM((CHUNK, BLK), jnp.int32),
                              pltpu.VMEM((CHUNK, BLK), jnp.int32),
                              pltpu.VMEM((BLK, D), jnp.float32),
                              pltpu.VMEM((BLK, D), jnp.float32),
                              pltpu.SemaphoreType.DMA,
                              pltpu.SemaphoreType.DMA,
                              pltpu.VMEM_SHARED((N_PAD, D), jnp.float32)))
    def k(h_hbm, src_hbm, dst_hbm, z_hbm, out_hbm, src_i, dst_i,
          rows_a, rows_b, sem_a, sem_b, agg_sh):
        cid = lax.axis_index("c")
        sid = lax.axis_index("s")
        row0 = sid * ROWS_PER_SUBCORE
        pltpu.sync_copy(z_hbm, agg_sh.at[pl.ds(row0, ROWS_PER_SUBCORE)])
        plsc.subcore_barrier()

        def start(b, rows, sem):
            pltpu.async_copy(h_hbm.at[src_i.at[b]], rows, sem)

        def finish(b, rows, sem):
            pltpu.make_async_copy(h_hbm.at[src_i.at[b]], rows, sem).wait()
            pltpu.sync_copy(rows, agg_sh.at[dst_i.at[b]], add=True)

        def run_chunk(blk0):
            pltpu.sync_copy(src_hbm.at[pl.ds(blk0, CHUNK)], src_i)
            pltpu.sync_copy(dst_hbm.at[pl.ds(blk0, CHUNK)], dst_i)
            start(0, rows_a, sem_a)

            @pl.loop(0, CHUNK // 2 - 1)
            def _(i):
                b0 = 2 * i
                start(b0 + 1, rows_b, sem_b)
                finish(b0, rows_a, sem_a)
                start(b0 + 2, rows_a, sem_a)
                finish(b0 + 1, rows_b, sem_b)

            start(CHUNK - 1, rows_b, sem_b)
            finish(CHUNK - 2, rows_a, sem_a)
            finish(CHUNK - 1, rows_b, sem_b)

        @pl.when(cid == 0)
        def _():
            for i in range(K0 // CHUNK):
                run_chunk(sid * K0 + i * CHUNK)

        @pl.when(cid == 1)
        def _():
            for i in range(K1 // CHUNK):
                run_chunk(NS * K0 + sid * K1 + i * CHUNK)

        plsc.subcore_barrier()
        pltpu.sync_copy(agg_sh.at[pl.ds(row0, ROWS_PER_SUBCORE)],
                        out_hbm.at[cid, pl.ds(row0, ROWS_PER_SUBCORE)])

    return k(h, src_p, dst_p, zeros_rows)


def _sc_degree(dst_p, ones_rows, zeros_rows):
    @pl.kernel(out_type=jax.ShapeDtypeStruct((NC, N_PAD, D), jnp.float32),
               mesh=_MESH,
               scratch_types=(pltpu.VMEM((BLOCKS_PER_WORKER, BLK), jnp.int32),
                              pltpu.VMEM((BLK, D), jnp.float32),
                              pltpu.VMEM_SHARED((N_PAD, D), jnp.float32)))
    def k(dst_hbm, ones_hbm, z_hbm, out_hbm, dst_i, ones_v, deg_sh):
        cid = lax.axis_index("c")
        sid = lax.axis_index("s")
        wid = sid * NC + cid
        row0 = sid * ROWS_PER_SUBCORE
        pltpu.sync_copy(z_hbm, deg_sh.at[pl.ds(row0, ROWS_PER_SUBCORE)])
        pltpu.sync_copy(ones_hbm, ones_v)
        blk0 = wid * BLOCKS_PER_WORKER
        pltpu.sync_copy(dst_hbm.at[pl.ds(blk0, BLOCKS_PER_WORKER)], dst_i)
        plsc.subcore_barrier()

        @pl.loop(0, BLOCKS_PER_WORKER)
        def _(b):
            pltpu.sync_copy(ones_v, deg_sh.at[dst_i.at[b]], add=True)

        plsc.subcore_barrier()
        pltpu.sync_copy(deg_sh.at[pl.ds(row0, ROWS_PER_SUBCORE)],
                        out_hbm.at[cid, pl.ds(row0, ROWS_PER_SUBCORE)])

    return k(dst_p, ones_rows, zeros_rows)


ROW_BLK = 1000


def _tc_layer_body(apply_relu, x_ref, a_ref, d_ref, ws_ref, wn_ref, b_ref, o_ref):
    inv = 1.0 / jnp.maximum(d_ref[:, 0] + d_ref[:, 1], 1.0)
    h_neigh = (a_ref[0] + a_ref[1]) * inv[:, None]
    dn = (((1,), (1,)), ((), ()))
    out = (lax.dot_general(x_ref[...], ws_ref[...], dn,
                           preferred_element_type=jnp.float32)
           + lax.dot_general(h_neigh, wn_ref[...], dn,
                             preferred_element_type=jnp.float32)
           + b_ref[...])
    if apply_relu:
        out = jnp.maximum(out, 0.0)
    o_ref[...] = out


def _tc_layer(x, agg_parts, deg, w_self, w_neigh, b, apply_relu):
    n = x.shape[0]
    grid = (n // ROW_BLK,)
    return pl.pallas_call(
        functools.partial(_tc_layer_body, apply_relu),
        grid=grid,
        in_specs=[
            pl.BlockSpec((ROW_BLK, D), lambda i: (i, 0)),
            pl.BlockSpec((NC, ROW_BLK, D), lambda i: (0, i, 0)),
            pl.BlockSpec((ROW_BLK, NC), lambda i: (i, 0)),
            pl.BlockSpec((D, D), lambda i: (0, 0)),
            pl.BlockSpec((D, D), lambda i: (0, 0)),
            pl.BlockSpec((1, D), lambda i: (0, 0)),
        ],
        out_specs=pl.BlockSpec((ROW_BLK, D), lambda i: (i, 0)),
        out_shape=jax.ShapeDtypeStruct((n, D), jnp.float32),
    )(x, agg_parts, deg, w_self, w_neigh, b)


def kernel(inputs, edge_index, W1_self, W1_neigh, b1, W2_self, W2_neigh, b2):
    x = inputs.astype(jnp.float32)
    src = edge_index[0].astype(jnp.int32)
    dst = edge_index[1].astype(jnp.int32)

    n_dummy = E_PAD - src.shape[0]
    src_p = jnp.concatenate([src, jnp.zeros((n_dummy,), jnp.int32)]).reshape(
        NW * BLOCKS_PER_WORKER, BLK)
    dst_p = jnp.concatenate(
        [dst, N_NODES + jnp.arange(n_dummy, dtype=jnp.int32) % (N_PAD - N_NODES)]
    ).reshape(NW * BLOCKS_PER_WORKER, BLK)

    zeros_rows = jnp.zeros((ROWS_PER_SUBCORE, D), jnp.float32)
    ones_rows = jnp.ones((BLK, D), jnp.float32)
    b1r = b1.reshape(1, D)
    b2r = b2.reshape(1, D)

    deg_parts = _sc_degree(dst_p, ones_rows, zeros_rows)
    deg = deg_parts[:, :, 0].T

    agg1 = _sc_aggregate(x, src_p, dst_p, zeros_rows)
    h = _tc_layer(x, agg1, deg, W1_self, W1_neigh, b1r, apply_relu=True)
    agg2 = _sc_aggregate(h, src_p, dst_p, zeros_rows)
    out = _tc_layer(h, agg2, deg, W2_self, W2_neigh, b2r, apply_relu=False)
    return out
